# baseline (speedup 1.0000x reference)
import functools

import jax
import jax.numpy as jnp
from jax.experimental import pallas as pl
from jax.experimental.pallas import tpu as pltpu
from jax.experimental.pallas import tpu_sc as plsc

_CUTOFF = 5.0
_LOG2 = 0.6931471805599453


def _ssp(v):
    return jnp.maximum(v, 0.0) + jnp.log1p(jnp.exp(-jnp.abs(v))) - _LOG2


def _in2f_body(x_ref, w_ref, o_ref):
    o_ref[...] = jnp.dot(x_ref[...].astype(jnp.bfloat16),
                         w_ref[...].astype(jnp.bfloat16),
                         preferred_element_type=jnp.float32)


def _in2f(x_flat, w):
    n, f = x_flat.shape[0], w.shape[1]
    return pl.pallas_call(
        _in2f_body,
        out_shape=jax.ShapeDtypeStruct((n, f), jnp.float32),
    )(x_flat, w)


def _sc_gather(table, idx, win=128):
    num_idx = idx.shape[1]
    fdim = table.shape[1]
    mesh = plsc.VectorSubcoreMesh(core_axis_name="c", subcore_axis_name="s")

    @functools.partial(
        pl.kernel,
        out_type=jax.ShapeDtypeStruct((num_idx, fdim), table.dtype),
        mesh=mesh,
    )
    def run(tab_hbm, i_hbm, o_hbm):
        def body(i_vmem, o_vmem):
            pltpu.sync_copy(tab_hbm.at[i_vmem.at[0]], o_vmem)

        pltpu.emit_pipeline(
            body,
            grid=(num_idx // win,),
            in_specs=[pl.BlockSpec((1, win), index_map=lambda i: (0, i))],
            out_specs=[pl.BlockSpec((win, fdim), index_map=lambda i: (i, 0))],
            core_axis_name=("c", "s"),
            dimension_semantics=(pltpu.PARALLEL,),
        )(i_hbm, o_hbm)

    return run(table, idx)


def _sc_gather_async(table, idx, win=128):
    num_idx = idx.shape[1]
    fdim = table.shape[1]
    nsteps = num_idx // (win * 32)
    mesh = plsc.VectorSubcoreMesh(core_axis_name="c", subcore_axis_name="s")

    @functools.partial(
        pl.kernel,
        out_type=jax.ShapeDtypeStruct((num_idx, fdim), table.dtype),
        mesh=mesh,
        scratch_types=[
            pltpu.VMEM((4, 1, win), jnp.int32),
            pltpu.VMEM((3, win, table.shape[1]), table.dtype),
            pltpu.SemaphoreType.DMA((4,)),
            pltpu.SemaphoreType.DMA((4,)),
            pltpu.SemaphoreType.DMA((3,)),
        ],
    )
    def run(tab_hbm, i_hbm, o_hbm, idx_buf, g_buf, sem_i, sem_g, sem_o):
        worker = jax.lax.axis_index("c") * 16 + jax.lax.axis_index("s")
        base = worker * nsteps

        def idx_dma(t, start):
            cp = pltpu.make_async_copy(
                i_hbm.at[:, pl.ds((base + t) * win, win)],
                idx_buf.at[t % 4],
                sem_i.at[t % 4])
            cp.start() if start else cp.wait()

        def gather(t, start):
            cp = pltpu.make_async_copy(
                tab_hbm.at[idx_buf.at[t % 4, 0]],
                g_buf.at[t % 3],
                sem_g.at[t % 4])
            cp.start() if start else cp.wait()

        def out(t, start):
            cp = pltpu.make_async_copy(
                g_buf.at[t % 3],
                o_hbm.at[pl.ds((base + t) * win, win), :],
                sem_o.at[t % 3])
            cp.start() if start else cp.wait()

        idx_dma(0, True)
        idx_dma(1, True)
        idx_dma(2, True)
        idx_dma(3, True)

        @pl.loop(0, nsteps)
        def _(t):
            @pl.when(t >= 2)
            def _():
                gather(t - 2, False)
                out(t - 2, True)

                @pl.when(t + 2 < nsteps)
                def _():
                    idx_dma(t + 2, True)

            @pl.when(t >= 3)
            def _():
                out(t - 3, False)

            idx_dma(t, False)
            gather(t, True)

        gather(nsteps - 2, False)
        out(nsteps - 2, True)
        gather(nsteps - 1, False)
        out(nsteps - 1, True)
        out(nsteps - 3, False)
        out(nsteps - 2, False)
        out(nsteps - 1, False)

    return run(table, idx)


def _filter_body(nn, ba, fij_ref, wf1_ref, bf1_ref, wf2_ref, bf2_ref, o_ref):
    s_dim = fij_ref.shape[-1]
    fij = fij_ref[...].reshape(ba * nn, s_dim)
    h = jnp.dot(fij, wf1_ref[...], preferred_element_type=jnp.float32)
    hb = (h + bf1_ref[...]).astype(jnp.bfloat16)
    hb = jnp.maximum(hb, jnp.bfloat16(0.0)) + jnp.log1p(
        jnp.exp(-jnp.abs(hb))) - jnp.bfloat16(_LOG2)
    o_ref[...] = jnp.dot(hb, wf2_ref[...],
                         preferred_element_type=jnp.float32) + bf2_ref[...]


def _agg_body(nn, ba, w_ref, yg_ref, wout_ref, bout_ref, wd_ref, bd_ref,
              o_ref):
    f = w_ref.shape[-1]
    s = w_ref[...] * yg_ref[...]
    v = jnp.sum(s.reshape(ba, nn, f), axis=1).astype(jnp.bfloat16)
    v = _ssp(jnp.dot(v, wout_ref[...], preferred_element_type=jnp.float32)
             + bout_ref[...]).astype(jnp.bfloat16)
    o_ref[...] = jnp.dot(v, wd_ref[...], preferred_element_type=jnp.float32) \
        + bd_ref[...]


def kernel(x, r_ij, neighbors, neighbor_mask, f_ij,
           W_in2f, W_f1, b_f1, W_f2, b_f2,
           W_out, b_out, W_dense, b_dense):
    B, Na, Nn = neighbors.shape
    S = f_ij.shape[-1]
    F = W_in2f.shape[1]

    zrow = B * Na
    x_pad = jnp.pad(x.reshape(B * Na, -1), ((0, 8), (0, 0)))
    y = _in2f(x_pad, W_in2f)

    nidx = B * Na * Nn
    pad_to = -(-nidx // (128 * 32)) * (128 * 32)
    live = (neighbor_mask != 0) & (r_ij <= _CUTOFF)
    flat_idx = jnp.where(
        live,
        neighbors + (jnp.arange(B, dtype=jnp.int32) * Na)[:, None, None],
        zrow).reshape(nidx)
    flat_idx = jnp.pad(flat_idx, (0, pad_to - nidx)).reshape(1, pad_to)
    yg = _sc_gather_async(y, flat_idx)

    BA = 400
    NB = (B * Na) // BA
    E = BA * Nn
    fij4 = f_ij.reshape(B * Na, Nn, S).astype(jnp.bfloat16)

    full = lambda shape: pl.BlockSpec(shape, lambda i: (0, 0))
    w_edges = pl.pallas_call(
        functools.partial(_filter_body, Nn, BA),
        grid=(NB,),
        in_specs=[
            pl.BlockSpec((BA, Nn, S), lambda i: (i, 0, 0)),
            full((S, F)),
            full((1, F)),
            full((F, F)),
            full((1, F)),
        ],
        out_specs=pl.BlockSpec((E, F), lambda i: (i, 0)),
        out_shape=jax.ShapeDtypeStruct((B * Na * Nn, F), jnp.float32),
        compiler_params=pltpu.CompilerParams(
            dimension_semantics=("parallel",)),
    )(fij4,
      W_f1.astype(jnp.bfloat16), b_f1.reshape(1, F),
      W_f2.astype(jnp.bfloat16), b_f2.reshape(1, F))

    out = pl.pallas_call(
        functools.partial(_agg_body, Nn, BA),
        grid=(NB,),
        in_specs=[
            pl.BlockSpec((E, F), lambda i: (i, 0)),
            pl.BlockSpec((E, F), lambda i: (i, 0)),
            full((F, F)),
            full((1, F)),
            full((F, F)),
            full((1, F)),
        ],
        out_specs=pl.BlockSpec((BA, F), lambda i: (i, 0)),
        out_shape=jax.ShapeDtypeStruct((B * Na, F), jnp.float32),
        compiler_params=pltpu.CompilerParams(
            dimension_semantics=("parallel",)),
    )(w_edges, yg,
      W_out.astype(jnp.bfloat16), b_out.reshape(1, F),
      W_dense.astype(jnp.bfloat16), b_dense.reshape(1, F))
    return out.reshape(B, Na, F)


# speedup vs baseline: 1.1375x; 1.1375x over previous
import functools

import jax
import jax.numpy as jnp
from jax.experimental import pallas as pl
from jax.experimental.pallas import tpu as pltpu
from jax.experimental.pallas import tpu_sc as plsc

_CUTOFF = 5.0
_LOG2 = 0.6931471805599453


def _ssp(v):
    return jnp.maximum(v, 0.0) + jnp.log1p(jnp.exp(-jnp.abs(v))) - _LOG2


def _in2f_body(x_ref, w_ref, o_ref):
    o_ref[...] = jnp.dot(x_ref[...].astype(jnp.bfloat16),
                         w_ref[...].astype(jnp.bfloat16),
                         preferred_element_type=jnp.float32)


def _in2f(x_flat, w):
    n, f = x_flat.shape[0], w.shape[1]
    return pl.pallas_call(
        _in2f_body,
        out_shape=jax.ShapeDtypeStruct((n, f), jnp.float32),
    )(x_flat, w)


def _sc_gather(table, idx, win=128):
    num_idx = idx.shape[1]
    fdim = table.shape[1]
    mesh = plsc.VectorSubcoreMesh(core_axis_name="c", subcore_axis_name="s")

    @functools.partial(
        pl.kernel,
        out_type=jax.ShapeDtypeStruct((num_idx, fdim), table.dtype),
        mesh=mesh,
    )
    def run(tab_hbm, i_hbm, o_hbm):
        def body(i_vmem, o_vmem):
            pltpu.sync_copy(tab_hbm.at[i_vmem.at[0]], o_vmem)

        pltpu.emit_pipeline(
            body,
            grid=(num_idx // win,),
            in_specs=[pl.BlockSpec((1, win), index_map=lambda i: (0, i))],
            out_specs=[pl.BlockSpec((win, fdim), index_map=lambda i: (i, 0))],
            core_axis_name=("c", "s"),
            dimension_semantics=(pltpu.PARALLEL,),
        )(i_hbm, o_hbm)

    return run(table, idx)


def _sc_gather_async(table, idx, win=128):
    num_idx = idx.shape[1]
    fdim = table.shape[1]
    nsteps = num_idx // (win * 32)
    mesh = plsc.VectorSubcoreMesh(core_axis_name="c", subcore_axis_name="s")

    @functools.partial(
        pl.kernel,
        out_type=jax.ShapeDtypeStruct((num_idx, fdim), table.dtype),
        mesh=mesh,
        scratch_types=[
            pltpu.VMEM((4, 1, win), jnp.int32),
            pltpu.VMEM((3, win, table.shape[1]), table.dtype),
            pltpu.SemaphoreType.DMA((4,)),
            pltpu.SemaphoreType.DMA((4,)),
            pltpu.SemaphoreType.DMA((3,)),
        ],
    )
    def run(tab_hbm, i_hbm, o_hbm, idx_buf, g_buf, sem_i, sem_g, sem_o):
        worker = jax.lax.axis_index("c") * 16 + jax.lax.axis_index("s")
        base = worker * nsteps

        def idx_dma(t, start):
            cp = pltpu.make_async_copy(
                i_hbm.at[:, pl.ds((base + t) * win, win)],
                idx_buf.at[t % 4],
                sem_i.at[t % 4])
            cp.start() if start else cp.wait()

        def gather(t, start):
            cp = pltpu.make_async_copy(
                tab_hbm.at[idx_buf.at[t % 4, 0]],
                g_buf.at[t % 3],
                sem_g.at[t % 4])
            cp.start() if start else cp.wait()

        def out(t, start):
            cp = pltpu.make_async_copy(
                g_buf.at[t % 3],
                o_hbm.at[pl.ds((base + t) * win, win), :],
                sem_o.at[t % 3])
            cp.start() if start else cp.wait()

        idx_dma(0, True)
        idx_dma(1, True)
        idx_dma(2, True)
        idx_dma(3, True)

        @pl.loop(0, nsteps)
        def _(t):
            @pl.when(t >= 2)
            def _():
                gather(t - 2, False)
                out(t - 2, True)

                @pl.when(t + 2 < nsteps)
                def _():
                    idx_dma(t + 2, True)

            @pl.when(t >= 3)
            def _():
                out(t - 3, False)

            idx_dma(t, False)
            gather(t, True)

        gather(nsteps - 2, False)
        out(nsteps - 2, True)
        gather(nsteps - 1, False)
        out(nsteps - 1, True)
        out(nsteps - 3, False)
        out(nsteps - 2, False)
        out(nsteps - 1, False)

    return run(table, idx)


def _fused_body(nn, ba, fij_ref, yg_ref,
                wf1_ref, bf1_ref, wf2_ref, bf2_ref,
                wout_ref, bout_ref, wd_ref, bd_ref, o_ref):
    f = wf1_ref.shape[1]
    s_dim = fij_ref.shape[-1]
    fij = fij_ref[...].reshape(ba * nn, s_dim)
    h = jnp.dot(fij, wf1_ref[...], preferred_element_type=jnp.float32)
    hb = (h + bf1_ref[...]).astype(jnp.bfloat16)
    hb = jnp.maximum(hb, jnp.bfloat16(0.0)) + jnp.log1p(
        jnp.exp(-jnp.abs(hb))) - jnp.bfloat16(_LOG2)
    w = jnp.dot(hb, wf2_ref[...],
                preferred_element_type=jnp.float32) + bf2_ref[...]
    s = w * yg_ref[...]
    v = jnp.sum(s.reshape(ba, nn, f), axis=1).astype(jnp.bfloat16)
    v = _ssp(jnp.dot(v, wout_ref[...], preferred_element_type=jnp.float32)
             + bout_ref[...]).astype(jnp.bfloat16)
    o_ref[...] = jnp.dot(v, wd_ref[...], preferred_element_type=jnp.float32) \
        + bd_ref[...]


def kernel(x, r_ij, neighbors, neighbor_mask, f_ij,
           W_in2f, W_f1, b_f1, W_f2, b_f2,
           W_out, b_out, W_dense, b_dense):
    B, Na, Nn = neighbors.shape
    S = f_ij.shape[-1]
    F = W_in2f.shape[1]

    zrow = B * Na
    x_pad = jnp.pad(x.reshape(B * Na, -1), ((0, 8), (0, 0)))
    y = _in2f(x_pad, W_in2f)

    nidx = B * Na * Nn
    pad_to = -(-nidx // (128 * 32)) * (128 * 32)
    live = (neighbor_mask != 0) & (r_ij <= _CUTOFF)
    flat_idx = jnp.where(
        live,
        neighbors + (jnp.arange(B, dtype=jnp.int32) * Na)[:, None, None],
        zrow).reshape(nidx)
    flat_idx = jnp.pad(flat_idx, (0, pad_to - nidx)).reshape(1, pad_to)
    yg = _sc_gather_async(y, flat_idx)

    BA = 400
    NB = (B * Na) // BA
    E = BA * Nn
    fij4 = f_ij.reshape(B * Na, Nn, S).astype(jnp.bfloat16)

    full = lambda shape: pl.BlockSpec(shape, lambda i: (0, 0))
    out = pl.pallas_call(
        functools.partial(_fused_body, Nn, BA),
        grid=(NB,),
        in_specs=[
            pl.BlockSpec((BA, Nn, S), lambda i: (i, 0, 0)),
            pl.BlockSpec((E, F), lambda i: (i, 0)),
            full((S, F)),
            full((1, F)),
            full((F, F)),
            full((1, F)),
            full((F, F)),
            full((1, F)),
            full((F, F)),
            full((1, F)),
        ],
        out_specs=pl.BlockSpec((BA, F), lambda i: (i, 0)),
        out_shape=jax.ShapeDtypeStruct((B * Na, F), jnp.float32),
        compiler_params=pltpu.CompilerParams(
            dimension_semantics=("parallel",)),
    )(fij4, yg,
      W_f1.astype(jnp.bfloat16), b_f1.reshape(1, F),
      W_f2.astype(jnp.bfloat16), b_f2.reshape(1, F),
      W_out.astype(jnp.bfloat16), b_out.reshape(1, F),
      W_dense.astype(jnp.bfloat16), b_dense.reshape(1, F))
    return out.reshape(B, Na, F)
